# baseline (speedup 1.0000x reference)
import functools

import jax
import jax.numpy as jnp
from jax import lax
from jax.experimental import pallas as pl
from jax.experimental.pallas import tpu as pltpu
from jax.experimental.pallas import tpu_sc as plsc

NC = 2
NS = 16
NT = NC * NS
CH = 128
NBUF = 4

F32 = jnp.float32


def _mm_body(x_ref, w_ref, h_ref):
    h_ref[...] = jnp.dot(x_ref[...], w_ref[...], preferred_element_type=F32)


def _scale_body(degp_ref, h_ref, dinv_ref, g_ref):
    dh = h_ref.shape[1] // 2
    deg = degp_ref[0, :, 0:1] + degp_ref[1, :, 0:1] + 1.0
    dinv = jnp.where(deg > 0, lax.rsqrt(deg), 0.0)
    dinv_ref[...] = dinv
    g_ref[...] = h_ref[...] * dinv


def _layer_body(s_ref, h_ref, dinv_ref, b_ref, w_ref, hn_ref, gn_ref):
    dh = h_ref.shape[1] // 2
    dinv = dinv_ref[...]
    h = h_ref[...]
    b = b_ref[...]
    d2 = dinv * dinv
    a_lo = jnp.maximum(dinv * s_ref[0] + d2 * h[:, :dh] + b[:, :dh], 0.0)
    a_hi = jnp.maximum(dinv * s_ref[1] + d2 * h[:, dh:] + b[:, dh:], 0.0)
    hn = (jnp.dot(a_lo, w_ref[:dh, :], preferred_element_type=F32)
          + jnp.dot(a_hi, w_ref[dh:, :], preferred_element_type=F32))
    hn_ref[...] = hn
    gn_ref[...] = hn * dinv


def _final_body(s_ref, h_ref, dinv_ref, b_ref, o_ref):
    dh = h_ref.shape[1] // 2
    dinv = dinv_ref[...]
    h = h_ref[...]
    b = b_ref[...]
    d2 = dinv * dinv
    o_ref[:, :dh] = jnp.maximum(
        dinv * s_ref[0] + d2 * h[:, :dh] + b[:, :dh], 0.0)
    o_ref[:, dh:] = jnp.maximum(
        dinv * s_ref[1] + d2 * h[:, dh:] + b[:, dh:], 0.0)


def _make_sc_kernels(n_pad, nch_deg, nch, dh):
    rows_pt = n_pad // NS
    mesh = plsc.VectorSubcoreMesh(
        core_axis_name="c", subcore_axis_name="s",
        num_cores=NC, num_subcores=NS)
    params = pltpu.CompilerParams(use_tc_tiling_on_sc=False)

    @functools.partial(
        pl.kernel,
        out_type=jax.ShapeDtypeStruct((NC, n_pad, 16), F32),
        mesh=mesh,
        compiler_params=params,
        scratch_types=[
            pltpu.VMEM((nch_deg, CH), jnp.int32),
            pltpu.VMEM((CH, 16), F32),
            pltpu.VMEM_SHARED((n_pad, 16), F32),
        ],
    )
    def deg_kernel(di_hbm, ones_hbm, zer_hbm, out_hbm, di_v, ones_v, acc):
        c = lax.axis_index("c")
        s = lax.axis_index("s")
        wid = c * NS + s
        pltpu.sync_copy(zer_hbm, acc.at[pl.ds(s * rows_pt, rows_pt)])
        pltpu.sync_copy(di_hbm.at[wid], di_v)
        pltpu.sync_copy(ones_hbm, ones_v)
        plsc.subcore_barrier()

        def body(j, carry):
            pltpu.sync_copy(ones_v, acc.at[di_v.at[j]], add=True)
            return carry

        lax.fori_loop(0, nch_deg, body, 0)
        plsc.subcore_barrier()
        pltpu.sync_copy(acc.at[pl.ds(s * rows_pt, rows_pt)],
                        out_hbm.at[c, pl.ds(s * rows_pt, rows_pt)])

    @functools.partial(
        pl.kernel,
        out_type=jax.ShapeDtypeStruct((NC, n_pad, dh), F32),
        mesh=mesh,
        compiler_params=params,
        scratch_types=[
            pltpu.VMEM((nch, CH), jnp.int32),
            pltpu.VMEM((nch, CH), jnp.int32),
            pltpu.VMEM((NBUF, CH, dh), F32),
            pltpu.VMEM_SHARED((n_pad, dh), F32),
        ] + [pltpu.SemaphoreType.DMA] * NBUF,
    )
    def gat_kernel(g_hbm, si_hbm, di_hbm, zer_hbm, out_hbm,
                   si_v, di_v, bufs, acc, *sems):
        c = lax.axis_index("c")
        s = lax.axis_index("s")
        wid = c * NS + s
        pltpu.sync_copy(zer_hbm, acc.at[pl.ds(s * rows_pt, rows_pt)])
        pltpu.sync_copy(si_hbm.at[wid], si_v)
        pltpu.sync_copy(di_hbm.at[s], di_v)
        plsc.subcore_barrier()
        for b in range(NBUF):
            pltpu.async_copy(g_hbm.at[si_v.at[b]], bufs.at[b], sems[b])

        def body(j, carry):
            for b in range(NBUF):
                jj = j * NBUF + b
                pltpu.make_async_copy(
                    g_hbm.at[si_v.at[jj]], bufs.at[b], sems[b]).wait()
                pltpu.sync_copy(bufs.at[b], acc.at[di_v.at[jj]], add=True)

                @pl.when(jj + NBUF < nch)
                def _():
                    pltpu.async_copy(
                        g_hbm.at[si_v.at[jj + NBUF]], bufs.at[b], sems[b])
            return carry

        lax.fori_loop(0, nch // NBUF, body, 0)
        plsc.subcore_barrier()
        pltpu.sync_copy(acc.at[pl.ds(s * rows_pt, rows_pt)],
                        out_hbm.at[c, pl.ds(s * rows_pt, rows_pt)])

    return deg_kernel, gat_kernel


def kernel(x, edge_index, W1, b1, W2, b2, W3, b3):
    n, d = x.shape
    dh = d // 2
    e = edge_index.shape[1]

    n_pad = -(-(n + 1) // (NS * 8)) * (NS * 8)
    nch = -(-(-(-e // (NS * CH))) // NBUF) * NBUF
    e_pad = NS * nch * CH
    nch_deg = nch // 2
    rows_pt = n_pad // NS

    si = edge_index[0].astype(jnp.int32)
    di = edge_index[1].astype(jnp.int32)
    pad = jnp.full((e_pad - e,), n, jnp.int32)
    si_p = jnp.concatenate([si, pad])
    di_p = jnp.concatenate([di, pad])
    si_t = si_p.reshape(NS, nch, CH) * 2
    si_e = jnp.concatenate([si_t, si_t + 1], axis=0)
    di_e = di_p.reshape(NS, nch, CH)
    di_g = di_p.reshape(NT, nch_deg, CH)

    x_pad = jnp.pad(x, ((0, n_pad - n), (0, 0)))
    ones16 = jnp.ones((CH, 16), F32)
    zer16 = jnp.zeros((rows_pt, 16), F32)
    zerd = jnp.zeros((rows_pt, dh), F32)
    b1r = b1.reshape(1, d)
    b2r = b2.reshape(1, d)
    b3r = b3.reshape(1, d)

    deg_kernel, gat_kernel = _make_sc_kernels(n_pad, nch_deg, nch, dh)

    mm = pl.pallas_call(
        _mm_body, out_shape=jax.ShapeDtypeStruct((n_pad, d), F32))
    scale = pl.pallas_call(
        _scale_body, out_shape=(jax.ShapeDtypeStruct((n_pad, 1), F32),
                                jax.ShapeDtypeStruct((n_pad, d), F32)))
    layer = pl.pallas_call(
        _layer_body, out_shape=(jax.ShapeDtypeStruct((n_pad, d), F32),
                                jax.ShapeDtypeStruct((n_pad, d), F32)))
    final = pl.pallas_call(
        _final_body, out_shape=jax.ShapeDtypeStruct((n_pad, d), F32))

    h1 = mm(x_pad, W1)
    degp = deg_kernel(di_g, ones16, zer16)
    dinv, g1 = scale(degp, h1)
    s1 = gat_kernel(g1.reshape(NC * n_pad, dh), si_e, di_e, zerd)
    h2, g2 = layer(s1, h1, dinv, b1r, W2)
    s2 = gat_kernel(g2.reshape(NC * n_pad, dh), si_e, di_e, zerd)
    h3, g3 = layer(s2, h2, dinv, b2r, W3)
    s3 = gat_kernel(g3.reshape(NC * n_pad, dh), si_e, di_e, zerd)
    out = final(s3, h3, dinv, b3r)
    return out[:n]


# speedup vs baseline: 1.3286x; 1.3286x over previous
import functools

import jax
import jax.numpy as jnp
from jax import lax
from jax.experimental import pallas as pl
from jax.experimental.pallas import tpu as pltpu
from jax.experimental.pallas import tpu_sc as plsc

NC = 2
NS = 16
NT = NC * NS
CH = 128
NBUF = 4

F32 = jnp.float32


def _first_body(x_ref, w_ref, degp_ref, h_ref, dinv_ref, g_ref):
    dh = x_ref.shape[1] // 2
    deg = degp_ref[0, :, 0:1] + degp_ref[1, :, 0:1] + 1.0
    dinv = jnp.where(deg > 0, lax.rsqrt(deg), 0.0)
    dinv_ref[...] = dinv
    h = jnp.dot(x_ref[...], w_ref[...], preferred_element_type=F32)
    h_ref[...] = h
    g_ref[0] = h[:, :dh] * dinv
    g_ref[1] = h[:, dh:] * dinv


def _layer_body(s_ref, h_ref, dinv_ref, b_ref, w_ref, hn_ref, gn_ref):
    dh = h_ref.shape[1] // 2
    dinv = dinv_ref[...]
    h = h_ref[...]
    b = b_ref[...]
    d2 = dinv * dinv
    a_lo = jnp.maximum(dinv * s_ref[0] + d2 * h[:, :dh] + b[:, :dh], 0.0)
    a_hi = jnp.maximum(dinv * s_ref[1] + d2 * h[:, dh:] + b[:, dh:], 0.0)
    hn = (jnp.dot(a_lo, w_ref[:dh, :], preferred_element_type=F32)
          + jnp.dot(a_hi, w_ref[dh:, :], preferred_element_type=F32))
    hn_ref[...] = hn
    gn_ref[0] = hn[:, :dh] * dinv
    gn_ref[1] = hn[:, dh:] * dinv


def _final_body(s_ref, h_ref, dinv_ref, b_ref, o_ref):
    dh = h_ref.shape[1] // 2
    dinv = dinv_ref[...]
    h = h_ref[...]
    b = b_ref[...]
    d2 = dinv * dinv
    o_ref[:, :dh] = jnp.maximum(
        dinv * s_ref[0] + d2 * h[:, :dh] + b[:, :dh], 0.0)
    o_ref[:, dh:] = jnp.maximum(
        dinv * s_ref[1] + d2 * h[:, dh:] + b[:, dh:], 0.0)


def _make_sc_kernels(n_pad, nch_deg, nch, dh):
    rows_pt = n_pad // NS
    mesh = plsc.VectorSubcoreMesh(
        core_axis_name="c", subcore_axis_name="s",
        num_cores=NC, num_subcores=NS)
    params = pltpu.CompilerParams(use_tc_tiling_on_sc=False)

    @functools.partial(
        pl.kernel,
        out_type=jax.ShapeDtypeStruct((NC, n_pad, 16), F32),
        mesh=mesh,
        compiler_params=params,
        scratch_types=[
            pltpu.VMEM((nch_deg, CH), jnp.int32),
            pltpu.VMEM((CH, 16), F32),
            pltpu.VMEM_SHARED((n_pad, 16), F32),
        ],
    )
    def deg_kernel(di_hbm, ones_hbm, zer_hbm, out_hbm, di_v, ones_v, acc):
        c = lax.axis_index("c")
        s = lax.axis_index("s")
        wid = c * NS + s
        pltpu.sync_copy(zer_hbm, acc.at[pl.ds(s * rows_pt, rows_pt)])
        pltpu.sync_copy(di_hbm.at[wid], di_v)
        pltpu.sync_copy(ones_hbm, ones_v)
        plsc.subcore_barrier()

        def body(j, carry):
            pltpu.sync_copy(ones_v, acc.at[di_v.at[j]], add=True)
            return carry

        lax.fori_loop(0, nch_deg, body, 0)
        plsc.subcore_barrier()
        pltpu.sync_copy(acc.at[pl.ds(s * rows_pt, rows_pt)],
                        out_hbm.at[c, pl.ds(s * rows_pt, rows_pt)])

    @functools.partial(
        pl.kernel,
        out_type=jax.ShapeDtypeStruct((NC, n_pad, dh), F32),
        mesh=mesh,
        compiler_params=params,
        scratch_types=[
            pltpu.VMEM((nch, CH), jnp.int32),
            pltpu.VMEM((nch, CH), jnp.int32),
            pltpu.VMEM((NBUF, CH, dh), F32),
            pltpu.VMEM_SHARED((n_pad, dh), F32),
        ] + [pltpu.SemaphoreType.DMA] * NBUF,
    )
    def gat_kernel(g_hbm, si_hbm, di_hbm, zer_hbm, out_hbm,
                   si_v, di_v, bufs, acc, *sems):
        c = lax.axis_index("c")
        s = lax.axis_index("s")
        wid = c * NS + s
        pltpu.sync_copy(zer_hbm, acc.at[pl.ds(s * rows_pt, rows_pt)])
        pltpu.sync_copy(si_hbm.at[wid], si_v)
        pltpu.sync_copy(di_hbm.at[s], di_v)
        plsc.subcore_barrier()
        for b in range(NBUF):
            pltpu.async_copy(g_hbm.at[si_v.at[b]], bufs.at[b], sems[b])

        def body(j, carry):
            for b in range(NBUF):
                jj = j * NBUF + b
                pltpu.make_async_copy(
                    g_hbm.at[si_v.at[jj]], bufs.at[b], sems[b]).wait()
                pltpu.sync_copy(bufs.at[b], acc.at[di_v.at[jj]], add=True)

                @pl.when(jj + NBUF < nch)
                def _():
                    pltpu.async_copy(
                        g_hbm.at[si_v.at[jj + NBUF]], bufs.at[b], sems[b])
            return carry

        lax.fori_loop(0, nch // NBUF, body, 0)
        plsc.subcore_barrier()
        pltpu.sync_copy(acc.at[pl.ds(s * rows_pt, rows_pt)],
                        out_hbm.at[c, pl.ds(s * rows_pt, rows_pt)])

    return deg_kernel, gat_kernel


def kernel(x, edge_index, W1, b1, W2, b2, W3, b3):
    n, d = x.shape
    dh = d // 2
    e = edge_index.shape[1]

    n_pad = -(-(n + 1) // (NS * 8)) * (NS * 8)
    nch = -(-(-(-e // (NS * CH))) // NBUF) * NBUF
    e_pad = NS * nch * CH
    nch_deg = nch // 2
    rows_pt = n_pad // NS

    si = edge_index[0].astype(jnp.int32)
    di = edge_index[1].astype(jnp.int32)
    pad = jnp.full((e_pad - e,), n, jnp.int32)
    si_p = jnp.concatenate([si, pad])
    di_p = jnp.concatenate([di, pad])
    si_t = si_p.reshape(NS, nch, CH)
    si_e = jnp.concatenate([si_t, si_t + n_pad], axis=0)
    di_e = di_p.reshape(NS, nch, CH)
    di_g = di_p.reshape(NT, nch_deg, CH)

    x_pad = jnp.pad(x, ((0, n_pad - n), (0, 0)))
    ones16 = jnp.ones((CH, 16), F32)
    zer16 = jnp.zeros((rows_pt, 16), F32)
    zerd = jnp.zeros((rows_pt, dh), F32)
    b1r = b1.reshape(1, d)
    b2r = b2.reshape(1, d)
    b3r = b3.reshape(1, d)

    deg_kernel, gat_kernel = _make_sc_kernels(n_pad, nch_deg, nch, dh)

    first = pl.pallas_call(
        _first_body, out_shape=(jax.ShapeDtypeStruct((n_pad, d), F32),
                                jax.ShapeDtypeStruct((n_pad, 1), F32),
                                jax.ShapeDtypeStruct((NC, n_pad, dh), F32)))
    layer = pl.pallas_call(
        _layer_body, out_shape=(jax.ShapeDtypeStruct((n_pad, d), F32),
                                jax.ShapeDtypeStruct((NC, n_pad, dh), F32)))
    final = pl.pallas_call(
        _final_body, out_shape=jax.ShapeDtypeStruct((n_pad, d), F32))

    degp = deg_kernel(di_g, ones16, zer16)
    h1, dinv, g1 = first(x_pad, W1, degp)
    s1 = gat_kernel(g1.reshape(NC * n_pad, dh), si_e, di_e, zerd)
    h2, g2 = layer(s1, h1, dinv, b1r, W2)
    s2 = gat_kernel(g2.reshape(NC * n_pad, dh), si_e, di_e, zerd)
    h3, g3 = layer(s2, h2, dinv, b2r, W3)
    s3 = gat_kernel(g3.reshape(NC * n_pad, dh), si_e, di_e, zerd)
    out = final(s3, h3, dinv, b3r)
    return out[:n]
